# baseline (speedup 1.0000x reference)
import functools

import jax
import jax.numpy as jnp
from jax import lax
from jax.experimental import pallas as pl
from jax.experimental.pallas import tpu as pltpu
from jax.experimental.pallas import tpu_sc as plsc

D = 64
CHUNK = 128
W = 8
EPS = 1e-5
LN_ROWS = 2048


@functools.lru_cache(maxsize=None)
def _build_gather(n_tok):
    info = plsc.get_sparse_core_info()
    nc, ns = info.num_cores, info.num_subcores
    nw = nc * ns
    per_w = n_tok // nw
    n_l = per_w // CHUNK
    assert n_tok == nw * per_w and per_w == n_l * CHUNK and n_l % W == 0
    mesh = plsc.VectorSubcoreMesh(core_axis_name="c", subcore_axis_name="s")

    def body(idx_hbm, table_hbm, out_hbm, idx_v, rows_v, sem_g, sem_s):
        wid = lax.axis_index("s") * nc + lax.axis_index("c")
        pltpu.sync_copy(idx_hbm.at[pl.ds(wid * n_l, n_l)], idx_v)
        c_base = wid * n_l

        def gather_cp(l, b):
            return pltpu.make_async_copy(
                table_hbm.at[idx_v.at[l]], rows_v.at[b], sem_g.at[b])

        def store_cp(l, b):
            return pltpu.make_async_copy(
                rows_v.at[b], out_hbm.at[c_base + l], sem_s.at[b])

        for b in range(W - 1):
            gather_cp(b, b).start()

        def loop_body(i, carry):
            for b in range(W):
                l = W * i + b
                gather_cp(l, b).wait()
                store_cp(l, b).start()
                ln = l + W - 1
                bn = (b + W - 1) % W
                if b == 0:
                    @pl.when(i >= 1)
                    def _():
                        store_cp(l - 1, bn).wait()
                    gather_cp(ln, bn).start()
                else:
                    @pl.when(i < n_l // W - 1)
                    def _():
                        store_cp(l - 1, bn).wait()
                        gather_cp(ln, bn).start()
            return carry

        lax.fori_loop(0, n_l // W, loop_body, 0)
        for b in range(W):
            store_cp(n_l - W + b, b).wait()

    return pl.kernel(
        body,
        out_type=jax.ShapeDtypeStruct((n_tok // CHUNK, CHUNK, D), jnp.float32),
        mesh=mesh,
        compiler_params=pltpu.CompilerParams(
            needs_layout_passes=False, use_tc_tiling_on_sc=False
        ),
        scratch_types=[
            pltpu.VMEM((n_l, CHUNK), jnp.int32),
            pltpu.VMEM((W, CHUNK, D), jnp.float32),
            pltpu.SemaphoreType.DMA((W,)),
            pltpu.SemaphoreType.DMA((W,)),
        ],
    )


def _ln_body(g2_ref, b2_ref, x_ref, o_ref):
    x = x_ref[...]
    xa = x[:, :D]
    xb = x[:, D:]
    ma = jnp.sum(xa, axis=1, keepdims=True) * (1.0 / D)
    mb = jnp.sum(xb, axis=1, keepdims=True) * (1.0 / D)
    va = jnp.sum(xa * xa, axis=1, keepdims=True) * (1.0 / D) - ma * ma
    vb = jnp.sum(xb * xb, axis=1, keepdims=True) * (1.0 / D) - mb * mb
    ia = lax.rsqrt(va + EPS)
    ib = lax.rsqrt(vb + EPS)
    n = x.shape[0]
    scale = jnp.concatenate(
        [jnp.broadcast_to(ia, (n, D)), jnp.broadcast_to(ib, (n, D))], axis=1)
    shift = jnp.concatenate(
        [jnp.broadcast_to(ma, (n, D)), jnp.broadcast_to(mb, (n, D))], axis=1)
    o_ref[...] = (x - shift) * scale * g2_ref[...] + b2_ref[...]


@functools.lru_cache(maxsize=None)
def _build_ln(n_rows):
    assert n_rows % LN_ROWS == 0
    return pl.pallas_call(
        _ln_body,
        grid=(n_rows // LN_ROWS,),
        in_specs=[
            pl.BlockSpec((1, 2 * D), lambda i: (0, 0)),
            pl.BlockSpec((1, 2 * D), lambda i: (0, 0)),
            pl.BlockSpec((LN_ROWS, 2 * D), lambda i: (i, 0)),
        ],
        out_specs=pl.BlockSpec((LN_ROWS, 2 * D), lambda i: (i, 0)),
        out_shape=jax.ShapeDtypeStruct((n_rows, 2 * D), jnp.float32),
    )


def kernel(x, table, gamma, beta):
    n_b, n_l = x.shape
    idx = x.reshape(-1)
    if idx.dtype != jnp.int32:
        idx = idx.astype(jnp.int32)
    idx2 = idx.reshape(-1, CHUNK)
    gathered = _build_gather(idx.size)(idx2, table)
    paired = gathered.reshape(idx.size // 2, 2 * D)
    g2 = jnp.tile(gamma, 2).reshape(1, 2 * D)
    b2 = jnp.tile(beta, 2).reshape(1, 2 * D)
    y = _build_ln(idx.size // 2)(g2, b2, paired)
    return y.reshape(n_b, n_l, D)
